# baseline (speedup 1.0000x reference)
import functools

import jax
import jax.numpy as jnp
from jax.experimental import pallas as pl
from jax.experimental.pallas import tpu as pltpu

NUM_EXPERTS = 8
TOP_K = 2
HIDDEN = 1024
BT = 2048


def _router_block(x_ref, w_ref, logits_ref, aff_ref, idx_ref):
    x = x_ref[...]
    w = w_ref[...]
    logits = jax.lax.dot_general(
        x, w,
        dimension_numbers=(((1,), (1,)), ((), ())),
        preferred_element_type=jnp.float32,
    )
    m = jnp.max(logits, axis=1, keepdims=True)
    e = jnp.exp(logits - m)
    s = jnp.sum(e, axis=1, keepdims=True)
    aff = e / s

    iota = jax.lax.broadcasted_iota(jnp.int32, aff.shape, 1)
    big = jnp.int32(NUM_EXPERTS)
    v1 = jnp.max(aff, axis=1, keepdims=True)
    idx1 = jnp.min(jnp.where(aff == v1, iota, big), axis=1, keepdims=True)
    aff2 = jnp.where(iota == idx1, -1.0, aff)
    v2 = jnp.max(aff2, axis=1, keepdims=True)
    idx2 = jnp.min(jnp.where(aff2 == v2, iota, big), axis=1, keepdims=True)

    logits_ref[...] = logits
    aff_ref[...] = aff
    idx_ref[...] = jnp.concatenate([idx1, idx2], axis=1)


@jax.jit
def _router(x, W):
    T = x.shape[0]
    grid = (T // BT,)
    return pl.pallas_call(
        _router_block,
        grid=grid,
        in_specs=[
            pl.BlockSpec((BT, HIDDEN), lambda i: (i, 0)),
            pl.BlockSpec((NUM_EXPERTS, HIDDEN), lambda i: (0, 0)),
        ],
        out_specs=[
            pl.BlockSpec((BT, NUM_EXPERTS), lambda i: (i, 0)),
            pl.BlockSpec((BT, NUM_EXPERTS), lambda i: (i, 0)),
            pl.BlockSpec((BT, TOP_K), lambda i: (i, 0)),
        ],
        out_shape=[
            jax.ShapeDtypeStruct((T, NUM_EXPERTS), jnp.float32),
            jax.ShapeDtypeStruct((T, NUM_EXPERTS), jnp.float32),
            jax.ShapeDtypeStruct((T, TOP_K), jnp.int32),
        ],
    )(x, W)


def kernel(hidden_states, W):
    B, S, H = hidden_states.shape
    x = hidden_states.reshape(B * S, H)
    logits, aff, idx = _router(x, W)
    return (logits, aff, idx)


# speedup vs baseline: 2.2358x; 2.2358x over previous
import jax
import jax.numpy as jnp
from jax.experimental import pallas as pl

NUM_EXPERTS = 8
TOP_K = 2
HIDDEN = 1024
BT = 2048


def _router_block(x_ref, w_ref, logits_ref, aff_ref, idx_ref):
    x = x_ref[...]
    w = w_ref[...]
    logits = jax.lax.dot_general(
        w, x,
        dimension_numbers=(((1,), (1,)), ((), ())),
        preferred_element_type=jnp.float32,
    )
    m = jnp.max(logits, axis=0, keepdims=True)
    e = jnp.exp(logits - m)
    s = jnp.sum(e, axis=0, keepdims=True)
    aff = e * (1.0 / s)

    iota = jax.lax.broadcasted_iota(jnp.int32, aff.shape, 0)
    big = jnp.int32(NUM_EXPERTS)
    v1 = jnp.max(aff, axis=0, keepdims=True)
    idx1 = jnp.min(jnp.where(aff == v1, iota, big), axis=0, keepdims=True)
    aff2 = jnp.where(iota == idx1, -1.0, aff)
    v2 = jnp.max(aff2, axis=0, keepdims=True)
    idx2 = jnp.min(jnp.where(aff2 == v2, iota, big), axis=0, keepdims=True)

    logits_ref[0] = logits
    aff_ref[0] = aff
    idx_ref[0] = jnp.concatenate([idx1, idx2], axis=0)


@jax.jit
def _router(x, W):
    T = x.shape[0]
    nblk = T // BT
    grid = (nblk,)
    logits_t, aff_t, idx_t = pl.pallas_call(
        _router_block,
        grid=grid,
        in_specs=[
            pl.BlockSpec((BT, HIDDEN), lambda i: (i, 0)),
            pl.BlockSpec((NUM_EXPERTS, HIDDEN), lambda i: (0, 0)),
        ],
        out_specs=[
            pl.BlockSpec((1, NUM_EXPERTS, BT), lambda i: (i, 0, 0)),
            pl.BlockSpec((1, NUM_EXPERTS, BT), lambda i: (i, 0, 0)),
            pl.BlockSpec((1, TOP_K, BT), lambda i: (i, 0, 0)),
        ],
        out_shape=[
            jax.ShapeDtypeStruct((nblk, NUM_EXPERTS, BT), jnp.float32),
            jax.ShapeDtypeStruct((nblk, NUM_EXPERTS, BT), jnp.float32),
            jax.ShapeDtypeStruct((nblk, TOP_K, BT), jnp.int32),
        ],
    )(x, W)
    logits = logits_t.transpose(0, 2, 1).reshape(T, NUM_EXPERTS)
    aff = aff_t.transpose(0, 2, 1).reshape(T, NUM_EXPERTS)
    idx = idx_t.transpose(0, 2, 1).reshape(T, TOP_K)
    return logits, aff, idx


def kernel(hidden_states, W):
    B, S, H = hidden_states.shape
    x = hidden_states.reshape(B * S, H)
    return _router(x, W)
